# Initial kernel scaffold; baseline (speedup 1.0000x reference)
import jax
import jax.numpy as jnp
from jax.experimental import pallas as pl


def kernel(prem_pred_idx, prem_arg_idx, concl_pred_idx, concl_arg_idx, pred_table, arg_table, W1, b1, W2, b2):
    raise NotImplementedError("write your pallas kernel here")


# speedup vs baseline: 9.6241x; 9.6241x over previous
import functools

import jax
import jax.numpy as jnp
from jax import lax
from jax.experimental import pallas as pl
from jax.experimental.pallas import tpu as pltpu
from jax.experimental.pallas import tpu_sc as plsc

B = 16384
P = 20
D = 128
NPRED = 64
NARG = 128
NROWS = 648
NIDX = 64
NW = 32
BPW = B // NW
L = 16
NC8 = D // L


def _fold_body(pred_ref, arg_ref, w1_ref, b1_ref, out_ref):
    pred = pred_ref[...]
    arg = arg_ref[...]
    w1 = w1_ref[...]
    s = jnp.float32(1.0 / P)
    parts = [
        jnp.dot(pred, w1[0 * D:1 * D], preferred_element_type=jnp.float32) * s,
        jnp.dot(arg, w1[1 * D:2 * D], preferred_element_type=jnp.float32) * s,
        jnp.dot(arg, w1[2 * D:3 * D], preferred_element_type=jnp.float32) * s,
        jnp.dot(pred, w1[3 * D:4 * D], preferred_element_type=jnp.float32),
        jnp.dot(arg, w1[4 * D:5 * D], preferred_element_type=jnp.float32),
        jnp.dot(arg, w1[5 * D:6 * D], preferred_element_type=jnp.float32),
        b1_ref[...][None, :],
        jnp.zeros((7, D), jnp.float32),
    ]
    out_ref[...] = jnp.concatenate(parts, axis=0)


@functools.partial(
    pl.kernel,
    mesh=plsc.VectorSubcoreMesh(core_axis_name="c", subcore_axis_name="s"),
    out_type=jax.ShapeDtypeStruct((B,), jnp.float32),
    compiler_params=pltpu.CompilerParams(needs_layout_passes=False),
    scratch_types=[
        pltpu.VMEM((NROWS * D,), jnp.float32),
        pltpu.VMEM((BPW * NIDX,), jnp.int32),
        pltpu.VMEM((D,), jnp.float32),
        pltpu.VMEM((L,), jnp.float32),
        pltpu.VMEM((BPW,), jnp.float32),
    ],
)
def _sc_gather(table_hbm, cidx_hbm, w2_hbm, b2_hbm, out_hbm,
               table_v, cidx_v, w2_v, b2_v, out_v):
    wid = lax.axis_index("s") * 2 + lax.axis_index("c")
    base = wid * BPW
    pltpu.sync_copy(table_hbm, table_v)
    pltpu.sync_copy(cidx_hbm.at[pl.ds(base * NIDX, BPW * NIDX)], cidx_v)
    pltpu.sync_copy(w2_hbm, w2_v)
    pltpu.sync_copy(b2_hbm, b2_v)

    col = [lax.iota(jnp.int32, L) + (L * c) for c in range(NC8)]
    w2c = [w2_v[pl.ds(L * c, L)] for c in range(NC8)]
    mask0 = lax.iota(jnp.int32, L) == 0
    zero = jnp.zeros((L,), jnp.float32)
    b2s = b2_v[...][0]

    def body(b, carry):
        acc = [zero] * NC8
        for k in range(NIDX // L):
            iv = cidx_v[pl.ds(b * NIDX + L * k, L)]
            for j in range(L):
                r = jnp.full((L,), iv[j], jnp.int32)
                for c in range(NC8):
                    acc[c] = acc[c] + plsc.load_gather(table_v, [r + col[c]])
        sv = zero
        for c in range(NC8):
            sv = sv + jnp.maximum(acc[c], 0.0) * w2c[c]
        tot = jnp.sum(sv) + b2s
        sig = 1.0 / (1.0 + jnp.exp(jnp.full((L,), -tot)))
        plsc.store_scatter(out_v, [jnp.full((L,), b, jnp.int32)], sig, mask=mask0)
        return carry

    lax.fori_loop(0, BPW, body, 0)
    pltpu.sync_copy(out_v, out_hbm.at[pl.ds(base, BPW)])


def kernel(prem_pred_idx, prem_arg_idx, concl_pred_idx, concl_arg_idx,
           pred_table, arg_table, W1, b1, W2, b2):
    pp = prem_pred_idx.astype(jnp.int32)
    pa = prem_arg_idx.astype(jnp.int32)
    cp = concl_pred_idx.astype(jnp.int32)
    ca = concl_arg_idx.astype(jnp.int32)

    folded = pl.pallas_call(
        _fold_body,
        out_shape=jax.ShapeDtypeStruct((NROWS, D), jnp.float32),
    )(pred_table, arg_table, W1, b1)

    cidx = jnp.concatenate([
        pp,
        pa[:, :, 0] + NPRED,
        pa[:, :, 1] + (NPRED + NARG),
        cp[:, None] + (NPRED + 2 * NARG),
        ca[:, 0:1] + (2 * NPRED + 2 * NARG),
        ca[:, 1:2] + (2 * NPRED + 3 * NARG),
        jnp.full((B, 1), 2 * NPRED + 4 * NARG, jnp.int32),
    ], axis=1) * D

    out_flat = _sc_gather(folded.reshape(-1), cidx.reshape(-1), W2.reshape(-1),
                          jnp.pad(b2, (0, L - 1)))
    return out_flat.reshape(B, 1)


# speedup vs baseline: 25.8254x; 2.6834x over previous
import functools

import jax
import jax.numpy as jnp
from jax import lax
from jax.experimental import pallas as pl
from jax.experimental.pallas import tpu as pltpu
from jax.experimental.pallas import tpu_sc as plsc

B = 16384
P = 20
D = 128
NPRED = 64
NARG = 128
NROWS = 648
NIDX = 64
NW = 32
BPW = B // NW
L = 16
DW = D // 2
NCH = D // (2 * L)


def _fold_body(pred_ref, arg_ref, w1_ref, b1_ref, out_ref):
    pred = pred_ref[...]
    arg = arg_ref[...]
    w1 = w1_ref[...]
    s = jnp.float32(1.0 / P)
    parts = [
        jnp.dot(pred, w1[0 * D:1 * D], preferred_element_type=jnp.float32) * s,
        jnp.dot(arg, w1[1 * D:2 * D], preferred_element_type=jnp.float32) * s,
        jnp.dot(arg, w1[2 * D:3 * D], preferred_element_type=jnp.float32) * s,
        jnp.dot(pred, w1[3 * D:4 * D], preferred_element_type=jnp.float32),
        jnp.dot(arg, w1[4 * D:5 * D], preferred_element_type=jnp.float32),
        jnp.dot(arg, w1[5 * D:6 * D], preferred_element_type=jnp.float32),
        b1_ref[...][None, :],
        jnp.zeros((7, D), jnp.float32),
    ]
    out_ref[...] = jnp.concatenate(parts, axis=0)


def _pack_pairs(x_f32):
    xb = x_f32.astype(jnp.bfloat16)
    return lax.bitcast_convert_type(
        xb.reshape(xb.shape[:-1] + (xb.shape[-1] // 2, 2)), jnp.int32)


@functools.partial(
    pl.kernel,
    mesh=plsc.VectorSubcoreMesh(core_axis_name="c", subcore_axis_name="s"),
    out_type=jax.ShapeDtypeStruct((B,), jnp.float32),
    compiler_params=pltpu.CompilerParams(needs_layout_passes=False),
    scratch_types=[
        pltpu.VMEM((NROWS * DW,), jnp.int32),
        pltpu.VMEM((BPW * NIDX,), jnp.int32),
        pltpu.VMEM((DW,), jnp.int32),
        pltpu.VMEM((L,), jnp.float32),
        pltpu.VMEM((BPW,), jnp.float32),
    ],
)
def _sc_gather(table_hbm, cidx_hbm, w2_hbm, b2_hbm, out_hbm,
               table_v, cidx_v, w2_v, b2_v, out_v):
    wid = lax.axis_index("s") * 2 + lax.axis_index("c")
    base = wid * BPW
    pltpu.sync_copy(table_hbm, table_v)
    pltpu.sync_copy(cidx_hbm.at[pl.ds(base * NIDX, BPW * NIDX)], cidx_v)
    pltpu.sync_copy(w2_hbm, w2_v)
    pltpu.sync_copy(b2_hbm, b2_v)

    col = [lax.iota(jnp.int32, L) + (L * c) for c in range(NCH)]
    w2b = [plsc.bitcast(w2_v[pl.ds(L * c, L)], jnp.bfloat16) for c in range(NCH)]
    mask0 = lax.iota(jnp.int32, L) == 0
    zero32 = jnp.zeros((2 * L,), jnp.bfloat16)
    b2s = b2_v[...][0]

    def body(b, carry):
        def chunk(k, accs):
            accs = list(accs)
            iv = cidx_v[pl.ds(b * NIDX + L * k, L)]
            for j in range(L):
                r = jnp.full((L,), iv[j], jnp.int32)
                for c in range(NCH):
                    w = plsc.load_gather(table_v, [r + col[c]])
                    accs[c] = accs[c] + plsc.bitcast(w, jnp.bfloat16)
            return tuple(accs)

        accs = lax.fori_loop(0, NIDX // L, chunk, (zero32,) * NCH)
        sv = zero32
        for c in range(NCH):
            sv = sv + jnp.maximum(accs[c], 0) * w2b[c]
        lo, hi = plsc.unpack(sv, format=plsc.PackFormat.INTERLEAVED)
        tot = jnp.sum(lo + hi) + b2s
        sig = 1.0 / (1.0 + jnp.exp(jnp.full((L,), -tot)))
        plsc.store_scatter(out_v, [jnp.full((L,), b, jnp.int32)], sig, mask=mask0)
        return carry

    lax.fori_loop(0, BPW, body, 0)
    pltpu.sync_copy(out_v, out_hbm.at[pl.ds(base, BPW)])


def kernel(prem_pred_idx, prem_arg_idx, concl_pred_idx, concl_arg_idx,
           pred_table, arg_table, W1, b1, W2, b2):
    pp = prem_pred_idx.astype(jnp.int32)
    pa = prem_arg_idx.astype(jnp.int32)
    cp = concl_pred_idx.astype(jnp.int32)
    ca = concl_arg_idx.astype(jnp.int32)

    folded = pl.pallas_call(
        _fold_body,
        out_shape=jax.ShapeDtypeStruct((NROWS, D), jnp.float32),
    )(pred_table, arg_table, W1, b1)

    cidx = jnp.concatenate([
        pp,
        pa[:, :, 0] + NPRED,
        pa[:, :, 1] + (NPRED + NARG),
        cp[:, None] + (NPRED + 2 * NARG),
        ca[:, 0:1] + (2 * NPRED + 2 * NARG),
        ca[:, 1:2] + (2 * NPRED + 3 * NARG),
        jnp.full((B, 1), 2 * NPRED + 4 * NARG, jnp.int32),
    ], axis=1) * DW

    out_flat = _sc_gather(_pack_pairs(folded).reshape(-1), cidx.reshape(-1),
                          _pack_pairs(W2.reshape(1, D)).reshape(-1),
                          jnp.pad(b2, (0, L - 1)))
    return out_flat.reshape(B, 1)
